# baseline (speedup 1.0000x reference)
import jax
import jax.numpy as jnp
from jax.experimental import pallas as pl
from jax.experimental.pallas import tpu as pltpu

N_NODES = 10000
N_EDGES = 160000
D = 256
EB = 800
NB = N_EDGES // EB
HALF_BLK = NB // 2
NBLK = 200
NNB = N_NODES // NBLK


def _edge_kernel(src_ref, typ_ref, dst_ref, nrm_ref, cw_ref,
                 x_ref, rel_ref, inw_ref, outw_ref,
                 sum_ref, max_ref, cnt_ref,
                 comp_ref, msg_ref):
    k = pl.program_id(0)

    @pl.when(k == 0)
    def _init():
        sum_ref[...] = jnp.zeros_like(sum_ref)
        max_ref[...] = jnp.full_like(max_ref, -jnp.inf)
        cnt_ref[...] = jnp.zeros_like(cnt_ref)

    a = cw_ref[0, 0] + cw_ref[0, 2]
    b = cw_ref[0, 1]
    c = cw_ref[0, 2] - cw_ref[0, 0]

    def gather_body(i, carry):
        s = src_ref[0, 0, i]
        t = typ_ref[0, 0, i]
        xr = x_ref[pl.ds(s, 1), :]
        rr = rel_ref[pl.ds(t, 1), :]
        comp_ref[pl.ds(i, 1), :] = a * xr + b * (xr * rr) + c * rr
        return carry

    jax.lax.fori_loop(0, EB, gather_body, 0)

    m = (k < HALF_BLK).astype(jnp.float32)
    w = inw_ref[...] * m + outw_ref[...] * (1.0 - m)
    CH = 200
    for j in range(EB // CH):
        msg_ref[j * CH:(j + 1) * CH, :] = jnp.dot(
            comp_ref[j * CH:(j + 1) * CH, :], w,
            preferred_element_type=jnp.float32)

    def scatter_body(i, carry):
        d = dst_ref[0, 0, i]
        nrm = nrm_ref[0, 0, i]
        mrow = msg_ref[pl.ds(i, 1), :] * nrm
        sum_ref[pl.ds(d, 1), :] += mrow
        max_ref[pl.ds(d, 1), :] = jnp.maximum(max_ref[pl.ds(d, 1), :], mrow)
        cnt_ref[pl.ds(d, 1), :] += 1.0
        return carry

    jax.lax.fori_loop(0, EB, scatter_body, 0)


def _node_pre_kernel(cw_ref, aw_ref, cb_ref,
                     sum_ref, max_ref, cnt_ref, x_ref,
                     looprel_ref, loopw_ref, wmlp_ref, wcat_t_ref, wcat_b_ref,
                     bias_ref,
                     out_ref, stat_ref):
    nb = pl.program_id(0)

    @pl.when(nb == 0)
    def _init():
        stat_ref[...] = jnp.zeros_like(stat_ref)

    a = cw_ref[0, 0] + cw_ref[0, 2]
    b = cw_ref[0, 1]
    c = cw_ref[0, 2] - cw_ref[0, 0]

    cnt = cnt_ref[:, 0:1]
    seg_sum = sum_ref[...]
    seg_mean = seg_sum / jnp.maximum(cnt, 1.0)
    pos = cnt > 0.0
    seg_max = jnp.where(pos, max_ref[...], 0.0)
    h = (aw_ref[0, 0] * seg_sum + aw_ref[0, 1] * seg_mean
         + aw_ref[0, 2] * seg_max)
    xb = x_ref[...]
    h = jnp.where(pos, h, xb)

    lr = looprel_ref[...]
    loop_comp = a * xb + b * (xb * lr) + c * lr
    loop_msg = jnp.dot(loop_comp, loopw_ref[...],
                       preferred_element_type=jnp.float32)
    hsum = h + loop_msg
    comb = (cb_ref[0, 0] * hsum
            + cb_ref[0, 1] * jnp.dot(hsum, wmlp_ref[...],
                                     preferred_element_type=jnp.float32)
            + cb_ref[0, 2] * (jnp.dot(h, wcat_t_ref[...],
                                      preferred_element_type=jnp.float32)
                              + jnp.dot(loop_msg, wcat_b_ref[...],
                                        preferred_element_type=jnp.float32)))
    o = comb * (1.0 / 3.0) + bias_ref[...]
    out_ref[...] = o
    stat_ref[0:1, :] += jnp.sum(o, axis=0, keepdims=True)
    stat_ref[1:2, :] += jnp.sum(o * o, axis=0, keepdims=True)


def _node_bn_kernel(ac_ref, pre_ref, stat_ref, gamma_ref, beta_ref, out_ref):
    n = jnp.float32(N_NODES)
    mu = stat_ref[0:1, :] / n
    var = stat_ref[1:2, :] / n - mu * mu
    o = pre_ref[...]
    o = (o - mu) * jax.lax.rsqrt(var + 1e-5) * gamma_ref[...] + beta_ref[...]
    out_ref[...] = (ac_ref[0, 0] * o + ac_ref[0, 1] * jnp.maximum(o, 0.0)
                    + ac_ref[0, 2] * jnp.tanh(o))


def _rel_kernel(rel_ref, w_ref, out_ref):
    out_ref[...] = jnp.dot(rel_ref[...], w_ref[...],
                           preferred_element_type=jnp.float32)


def kernel(x, edge_index, rel_repr, edge_type, edge_norm, comp_weights,
           agg_weights, comb_weights, act_weights, in_w, out_w, loop_w,
           w_rel, loop_rel, bias, bn_gamma, bn_beta, w_mlp, w_cat):
    src3 = edge_index[0].reshape(NB, 1, EB)
    dst3 = edge_index[1].reshape(NB, 1, EB)
    typ3 = edge_type.reshape(NB, 1, EB)
    nrm3 = edge_norm.reshape(NB, 1, EB)
    cw = comp_weights.reshape(1, 3)
    aw = agg_weights.reshape(1, 3)
    cb = comb_weights.reshape(1, 3)
    ac = act_weights.reshape(1, 3)

    smem_idx = pl.BlockSpec((1, 1, EB), lambda k: (k, 0, 0),
                            memory_space=pltpu.SMEM)
    smem_w = pl.BlockSpec((1, 3), lambda k: (0, 0), memory_space=pltpu.SMEM)
    full = lambda s: pl.BlockSpec(s, lambda k: tuple(0 for _ in s))

    seg_sum, seg_max, cnt = pl.pallas_call(
        _edge_kernel,
        grid=(NB,),
        in_specs=[smem_idx, smem_idx, smem_idx, smem_idx, smem_w,
                  full((N_NODES, D)), full((200, D)),
                  full((D, D)), full((D, D))],
        out_specs=[full((N_NODES, D)), full((N_NODES, D)),
                   full((N_NODES, 128))],
        out_shape=[jax.ShapeDtypeStruct((N_NODES, D), jnp.float32),
                   jax.ShapeDtypeStruct((N_NODES, D), jnp.float32),
                   jax.ShapeDtypeStruct((N_NODES, 128), jnp.float32)],
        scratch_shapes=[pltpu.VMEM((EB, D), jnp.float32),
                        pltpu.VMEM((EB, D), jnp.float32)],
    )(src3, typ3, dst3, nrm3, cw, x, rel_repr, in_w, out_w)

    nblk = lambda s: pl.BlockSpec(s, lambda nb: (nb,) + (0,) * (len(s) - 1))
    nfull = lambda s: pl.BlockSpec(s, lambda nb: tuple(0 for _ in s))
    nsmem = pl.BlockSpec((1, 3), lambda nb: (0, 0), memory_space=pltpu.SMEM)

    out_pre, stat = pl.pallas_call(
        _node_pre_kernel,
        grid=(NNB,),
        in_specs=[nsmem, nsmem, nsmem,
                  nblk((NBLK, D)), nblk((NBLK, D)), nblk((NBLK, 128)),
                  nblk((NBLK, D)),
                  nfull((1, D)), nfull((D, D)), nfull((D, D)),
                  nfull((D, D)), nfull((D, D)),
                  nfull((1, D))],
        out_specs=[nblk((NBLK, D)), nfull((8, D))],
        out_shape=[jax.ShapeDtypeStruct((N_NODES, D), jnp.float32),
                   jax.ShapeDtypeStruct((8, D), jnp.float32)],
    )(cw, aw, cb, seg_sum, seg_max, cnt, x,
      loop_rel, loop_w, w_mlp, w_cat[:D], w_cat[D:], bias.reshape(1, D))

    out = pl.pallas_call(
        _node_bn_kernel,
        grid=(NNB,),
        in_specs=[nsmem, nblk((NBLK, D)), nfull((8, D)),
                  nfull((1, D)), nfull((1, D))],
        out_specs=nblk((NBLK, D)),
        out_shape=jax.ShapeDtypeStruct((N_NODES, D), jnp.float32),
    )(ac, out_pre, stat, bn_gamma.reshape(1, D), bn_beta.reshape(1, D))

    rel_out = pl.pallas_call(
        _rel_kernel,
        in_specs=[pl.BlockSpec((200, D), lambda: (0, 0)),
                  pl.BlockSpec((D, D), lambda: (0, 0))],
        out_specs=pl.BlockSpec((200, D), lambda: (0, 0)),
        out_shape=jax.ShapeDtypeStruct((200, D), jnp.float32),
    )(rel_repr, w_rel)

    return (out, rel_out)


# speedup vs baseline: 1.2686x; 1.2686x over previous
import jax
import jax.numpy as jnp
from jax.experimental import pallas as pl
from jax.experimental.pallas import tpu as pltpu

N_NODES = 10000
N_EDGES = 160000
D = 256
EB = 800
NB = N_EDGES // EB
HALF_BLK = NB // 2
NBLK = 200
NNB = N_NODES // NBLK


def _edge_kernel(src_ref, typ_ref, dst_ref, nrm_ref, cw_ref,
                 x_ref, rel_ref, inw_ref, outw_ref,
                 sum_ref, max_ref, cnt_ref,
                 comp_ref, msg_ref):
    k = pl.program_id(0)

    @pl.when(k == 0)
    def _init():
        sum_ref[...] = jnp.zeros_like(sum_ref)
        max_ref[...] = jnp.full_like(max_ref, -jnp.inf)
        cnt_ref[...] = jnp.zeros_like(cnt_ref)

    a = cw_ref[0, 0] + cw_ref[0, 2]
    b = cw_ref[0, 1]
    c = cw_ref[0, 2] - cw_ref[0, 0]

    def gather_one(i):
        s = src_ref[0, 0, i]
        t = typ_ref[0, 0, i]
        nrm = nrm_ref[0, 0, i]
        xr = x_ref[pl.ds(s, 1), :]
        rr = rel_ref[pl.ds(t, 1), :]
        comp_ref[pl.ds(i, 1), :] = (a * xr + b * (xr * rr) + c * rr) * nrm

    def gather_body(j, carry):
        gather_one(2 * j)
        gather_one(2 * j + 1)
        return carry

    jax.lax.fori_loop(0, EB // 2, gather_body, 0)

    m = (k < HALF_BLK).astype(jnp.float32)
    w = inw_ref[...] * m + outw_ref[...] * (1.0 - m)
    CH = 200
    for j in range(EB // CH):
        msg_ref[j * CH:(j + 1) * CH, :] = jnp.dot(
            comp_ref[j * CH:(j + 1) * CH, :], w,
            preferred_element_type=jnp.float32)

    def scatter_one(i):
        d = dst_ref[0, 0, i]
        mrow = msg_ref[pl.ds(i, 1), :]
        sum_ref[pl.ds(d, 1), :] += mrow
        max_ref[pl.ds(d, 1), :] = jnp.maximum(max_ref[pl.ds(d, 1), :], mrow)
        cnt_ref[pl.ds(d, 1), :] += 1.0

    def scatter_body(j, carry):
        scatter_one(2 * j)
        scatter_one(2 * j + 1)
        return carry

    jax.lax.fori_loop(0, EB // 2, scatter_body, 0)


def _node_pre_kernel(cw_ref, aw_ref, cb_ref,
                     sum_ref, max_ref, cnt_ref, x_ref,
                     looprel_ref, loopw_ref, wmlp_ref, wcat_t_ref, wcat_b_ref,
                     bias_ref,
                     out_ref, stat_ref):
    nb = pl.program_id(0)

    @pl.when(nb == 0)
    def _init():
        stat_ref[...] = jnp.zeros_like(stat_ref)

    a = cw_ref[0, 0] + cw_ref[0, 2]
    b = cw_ref[0, 1]
    c = cw_ref[0, 2] - cw_ref[0, 0]

    cnt = cnt_ref[:, 0:1]
    seg_sum = sum_ref[...]
    seg_mean = seg_sum / jnp.maximum(cnt, 1.0)
    pos = cnt > 0.0
    seg_max = jnp.where(pos, max_ref[...], 0.0)
    h = (aw_ref[0, 0] * seg_sum + aw_ref[0, 1] * seg_mean
         + aw_ref[0, 2] * seg_max)
    xb = x_ref[...]
    h = jnp.where(pos, h, xb)

    lr = looprel_ref[...]
    loop_comp = a * xb + b * (xb * lr) + c * lr
    loop_msg = jnp.dot(loop_comp, loopw_ref[...],
                       preferred_element_type=jnp.float32)
    hsum = h + loop_msg
    comb = (cb_ref[0, 0] * hsum
            + cb_ref[0, 1] * jnp.dot(hsum, wmlp_ref[...],
                                     preferred_element_type=jnp.float32)
            + cb_ref[0, 2] * (jnp.dot(h, wcat_t_ref[...],
                                      preferred_element_type=jnp.float32)
                              + jnp.dot(loop_msg, wcat_b_ref[...],
                                        preferred_element_type=jnp.float32)))
    o = comb * (1.0 / 3.0) + bias_ref[...]
    out_ref[...] = o
    stat_ref[0:1, :] += jnp.sum(o, axis=0, keepdims=True)
    stat_ref[1:2, :] += jnp.sum(o * o, axis=0, keepdims=True)


def _node_bn_kernel(ac_ref, pre_ref, stat_ref, gamma_ref, beta_ref, out_ref):
    n = jnp.float32(N_NODES)
    mu = stat_ref[0:1, :] / n
    var = stat_ref[1:2, :] / n - mu * mu
    o = pre_ref[...]
    o = (o - mu) * jax.lax.rsqrt(var + 1e-5) * gamma_ref[...] + beta_ref[...]
    out_ref[...] = (ac_ref[0, 0] * o + ac_ref[0, 1] * jnp.maximum(o, 0.0)
                    + ac_ref[0, 2] * jnp.tanh(o))


def _rel_kernel(rel_ref, w_ref, out_ref):
    out_ref[...] = jnp.dot(rel_ref[...], w_ref[...],
                           preferred_element_type=jnp.float32)


def kernel(x, edge_index, rel_repr, edge_type, edge_norm, comp_weights,
           agg_weights, comb_weights, act_weights, in_w, out_w, loop_w,
           w_rel, loop_rel, bias, bn_gamma, bn_beta, w_mlp, w_cat):
    src3 = edge_index[0].reshape(NB, 1, EB)
    dst3 = edge_index[1].reshape(NB, 1, EB)
    typ3 = edge_type.reshape(NB, 1, EB)
    nrm3 = edge_norm.reshape(NB, 1, EB)
    cw = comp_weights.reshape(1, 3)
    aw = agg_weights.reshape(1, 3)
    cb = comb_weights.reshape(1, 3)
    ac = act_weights.reshape(1, 3)

    smem_idx = pl.BlockSpec((1, 1, EB), lambda k: (k, 0, 0),
                            memory_space=pltpu.SMEM)
    smem_w = pl.BlockSpec((1, 3), lambda k: (0, 0), memory_space=pltpu.SMEM)
    full = lambda s: pl.BlockSpec(s, lambda k: tuple(0 for _ in s))

    seg_sum, seg_max, cnt = pl.pallas_call(
        _edge_kernel,
        grid=(NB,),
        in_specs=[smem_idx, smem_idx, smem_idx, smem_idx, smem_w,
                  full((N_NODES, D)), full((200, D)),
                  full((D, D)), full((D, D))],
        out_specs=[full((N_NODES, D)), full((N_NODES, D)),
                   full((N_NODES, 128))],
        out_shape=[jax.ShapeDtypeStruct((N_NODES, D), jnp.float32),
                   jax.ShapeDtypeStruct((N_NODES, D), jnp.float32),
                   jax.ShapeDtypeStruct((N_NODES, 128), jnp.float32)],
        scratch_shapes=[pltpu.VMEM((EB, D), jnp.float32),
                        pltpu.VMEM((EB, D), jnp.float32)],
    )(src3, typ3, dst3, nrm3, cw, x, rel_repr, in_w, out_w)

    nblk = lambda s: pl.BlockSpec(s, lambda nb: (nb,) + (0,) * (len(s) - 1))
    nfull = lambda s: pl.BlockSpec(s, lambda nb: tuple(0 for _ in s))
    nsmem = pl.BlockSpec((1, 3), lambda nb: (0, 0), memory_space=pltpu.SMEM)

    out_pre, stat = pl.pallas_call(
        _node_pre_kernel,
        grid=(NNB,),
        in_specs=[nsmem, nsmem, nsmem,
                  nblk((NBLK, D)), nblk((NBLK, D)), nblk((NBLK, 128)),
                  nblk((NBLK, D)),
                  nfull((1, D)), nfull((D, D)), nfull((D, D)),
                  nfull((D, D)), nfull((D, D)),
                  nfull((1, D))],
        out_specs=[nblk((NBLK, D)), nfull((8, D))],
        out_shape=[jax.ShapeDtypeStruct((N_NODES, D), jnp.float32),
                   jax.ShapeDtypeStruct((8, D), jnp.float32)],
    )(cw, aw, cb, seg_sum, seg_max, cnt, x,
      loop_rel, loop_w, w_mlp, w_cat[:D], w_cat[D:], bias.reshape(1, D))

    out = pl.pallas_call(
        _node_bn_kernel,
        grid=(NNB,),
        in_specs=[nsmem, nblk((NBLK, D)), nfull((8, D)),
                  nfull((1, D)), nfull((1, D))],
        out_specs=nblk((NBLK, D)),
        out_shape=jax.ShapeDtypeStruct((N_NODES, D), jnp.float32),
    )(ac, out_pre, stat, bn_gamma.reshape(1, D), bn_beta.reshape(1, D))

    rel_out = pl.pallas_call(
        _rel_kernel,
        in_specs=[pl.BlockSpec((200, D), lambda: (0, 0)),
                  pl.BlockSpec((D, D), lambda: (0, 0))],
        out_specs=pl.BlockSpec((200, D), lambda: (0, 0)),
        out_shape=jax.ShapeDtypeStruct((200, D), jnp.float32),
    )(rel_repr, w_rel)

    return (out, rel_out)
